# baseline (speedup 1.0000x reference)
import functools

import jax
import jax.numpy as jnp
from jax import lax
from jax.experimental import pallas as pl
from jax.experimental.pallas import tpu as pltpu
from jax.experimental.pallas import tpu_sc as plsc

D_EMBED = 64
D_MODEL = 1024
BATCH = 4
SEQ = 2048
B = BATCH * SEQ
VOCAB_N = 1000000
PAIR_W = 128

NC, NS = 2, 16
NW = NC * NS
R_PER_W = VOCAB_N // NW
CHUNK = 1536
NRB = 21
NBUF = 4
LIMIT = 1000064
CLAMP_MAX = (LIMIT - CHUNK) // 128 * 128
WAVE = 384
NPOS = ((B + WAVE - 1) // WAVE) * WAVE

_mesh = plsc.VectorSubcoreMesh(
    core_axis_name="c", subcore_axis_name="s", num_cores=NC, num_subcores=NS
)

_LANES = 16


def _splat(x):
    return jnp.full((_LANES,), x, jnp.int32)


@functools.partial(
    pl.kernel,
    out_type=jax.ShapeDtypeStruct((B, PAIR_W), jnp.float32),
    mesh=_mesh,
    scratch_types=[
        pltpu.VMEM((B,), jnp.int32),
        pltpu.VMEM((B,), jnp.int32),
        pltpu.VMEM((NPOS,), jnp.int32),
        *[pltpu.VMEM((1, 8, CHUNK), jnp.float32) for _ in range(NBUF)],
        pltpu.VMEM((WAVE, PAIR_W), jnp.float32),
        *[pltpu.SemaphoreType.DMA for _ in range(NBUF + 1)],
    ],
    compiler_params=pltpu.CompilerParams(
        use_tc_tiling_on_sc=True,
        disable_bounds_checks=True,
        needs_layout_passes=False,
    ),
)
def _sc_gather(table3_hbm, ids_hbm, out_hbm, ids_v, trel_v, pos_v,
               stage_0, stage_1, stage_2, stage_3, rows_v,
               sem_0, sem_1, sem_2, sem_3, sem_o):
    stages = [stage_0, stage_1, stage_2, stage_3]
    sems = [sem_0, sem_1, sem_2, sem_3]
    wid = lax.axis_index("s") * NC + lax.axis_index("c")
    rr0 = (wid * R_PER_W) // 128 * 128
    rr1 = jnp.where(wid == NW - 1, VOCAB_N, ((wid + 1) * R_PER_W) // 128 * 128)
    iota = lax.iota(jnp.int32, _LANES)

    pltpu.sync_copy(ids_hbm, ids_v)

    def prefill(i, _):
        plsc.store_scatter(pos_v, [i * _LANES + iota], _splat(-1))
        return 0

    lax.fori_loop(0, NPOS // _LANES, prefill, 0)

    def scan(i, cursor):
        tok = plsc.load_gather(ids_v, [i * _LANES + iota])
        m = (tok >= rr0) & (tok < rr1)
        mi = m.astype(jnp.int32)
        p = cursor + plsc.cumsum(mi) - 1
        plsc.store_scatter(trel_v, [p], tok - rr0, mask=m)
        plsc.store_scatter(pos_v, [p], i * _LANES + iota, mask=m)
        return cursor + plsc.all_reduce_population_count(m)

    cursor = lax.fori_loop(0, B // _LANES, scan, _splat(0))
    n = jnp.max(cursor)
    waves = (n + WAVE - 1) // WAVE

    zeros16 = _splat(0)

    def wave_body(w, _):
        lo = w * WAVE

        NJ = 8 * NRB

        def src(j):
            tr = j // NRB
            r_lo_c = jnp.minimum(rr0 + (j % NRB) * CHUNK, CLAMP_MAX)
            return table3_hbm.at[pl.ds(tr, 1), :, pl.ds(r_lo_c, CHUNK)]

        def fire(j, stage, sem):
            pltpu.async_copy(src(j), stage, sem)

        def gather(j, stage):
            tr = j // NRB
            base_rel = jnp.minimum(rr0 + (j % NRB) * CHUNK, CLAMP_MAX) - rr0
            c0 = tr * 8

            def gv_body(v, _):
                le = plsc.load_gather(trel_v, [lo + v * _LANES + iota])
                idx = le - base_rel
                m = (le >= 0) & (idx >= 0) & (idx < CHUNK)
                idx = jnp.clip(idx, 0, CHUNK - 1)
                il = v * _LANES + iota
                for cp in range(8):
                    vals = plsc.load_gather(
                        stage, [zeros16, _splat(cp), idx], mask=m
                    )
                    plsc.store_scatter(
                        rows_v, [il, jnp.full((_LANES,), c0 + cp, jnp.int32)],
                        vals, mask=m,
                    )
                return 0

            lax.fori_loop(0, WAVE // _LANES, gv_body, 0)

        def wait(j, stage, sem):
            pltpu.make_async_copy(src(j), stage, sem).wait()

        for s_ in range(NBUF):
            fire(s_, stages[s_], sems[s_])

        def ring_body(k, _):
            for s_ in range(NBUF):
                j = NBUF * k + s_
                wait(j, stages[s_], sems[s_])
                gather(j, stages[s_])

                @pl.when(k < NJ // NBUF - 1)
                def _(j=j, s_=s_):
                    fire(j + NBUF, stages[s_], sems[s_])

            return 0

        lax.fori_loop(0, NJ // NBUF, ring_body, 0)

        pltpu.async_copy(
            rows_v,
            out_hbm.at[plsc.Indices(pos_v.at[pl.ds(lo, WAVE)], ignored_value=-1)],
            sem_o,
        ).wait()
        return 0

    lax.fori_loop(0, waves, wave_body, 0)


def _proj_body(x_ref, w2_ref, o_ref):
    res = lax.dot_general(
        x_ref[...], w2_ref[...],
        dimension_numbers=(((1,), (1,)), ((), ())),
        preferred_element_type=jnp.float32,
    )
    o_ref[...] = res.reshape(ROW_BLK // BATCH, BATCH, D_MODEL)


ROW_BLK = 1024


def _project(x, w2):
    return pl.pallas_call(
        _proj_body,
        grid=(B // ROW_BLK,),
        in_specs=[
            pl.BlockSpec((ROW_BLK, PAIR_W), lambda i: (i, 0)),
            pl.BlockSpec((D_MODEL, PAIR_W), lambda i: (0, 0)),
        ],
        out_specs=pl.BlockSpec(
            (ROW_BLK // BATCH, BATCH, D_MODEL), lambda i: (i, 0, 0)
        ),
        out_shape=jax.ShapeDtypeStruct((SEQ, BATCH, D_MODEL), jnp.float32),
    )(x, w2)


def kernel(input_ids, emb_table, emb_proj):
    ids = jnp.transpose(input_ids, (1, 0)).astype(jnp.int32).reshape(B)
    table3 = jnp.transpose(emb_table, (1, 0)).reshape(8, 8, VOCAB_N)
    w2 = jnp.concatenate(
        [emb_proj, jnp.zeros((D_MODEL, D_EMBED), jnp.float32)], axis=1
    )
    rows = _sc_gather(table3, ids)
    return _project(rows, w2)


# speedup vs baseline: 1.2859x; 1.2859x over previous
import functools

import jax
import jax.numpy as jnp
from jax import lax
from jax.experimental import pallas as pl
from jax.experimental.pallas import tpu as pltpu
from jax.experimental.pallas import tpu_sc as plsc

D_EMBED = 64
D_MODEL = 1024
BATCH = 4
SEQ = 2048
B = BATCH * SEQ
VOCAB_N = 1000000
PAIR_W = 128

NC, NS = 2, 16
NW = NC * NS
R_PER_W = VOCAB_N // NW
CHUNK = 2688
NRB = 12
LIMIT = 1000064
CLAMP_MAX = (LIMIT - CHUNK) // 128 * 128
WAVE = 384
NPOS = ((B + WAVE - 1) // WAVE) * WAVE

_mesh = plsc.VectorSubcoreMesh(
    core_axis_name="c", subcore_axis_name="s", num_cores=NC, num_subcores=NS
)

_LANES = 16


def _splat(x):
    return jnp.full((_LANES,), x, jnp.int32)


@functools.partial(
    pl.kernel,
    out_type=jax.ShapeDtypeStruct((B, PAIR_W), jnp.float32),
    mesh=_mesh,
    scratch_types=[
        pltpu.VMEM((B,), jnp.int32),
        pltpu.VMEM((B,), jnp.int32),
        pltpu.VMEM((NPOS,), jnp.int32),
        pltpu.VMEM((1, 8, CHUNK), jnp.float32),
        pltpu.VMEM((1, 8, CHUNK), jnp.float32),
        pltpu.VMEM((WAVE, PAIR_W), jnp.float32),
        pltpu.SemaphoreType.DMA,
        pltpu.SemaphoreType.DMA,
        pltpu.SemaphoreType.DMA,
    ],
    compiler_params=pltpu.CompilerParams(
        use_tc_tiling_on_sc=True,
        disable_bounds_checks=True,
        needs_layout_passes=False,
    ),
)
def _sc_gather(table3_hbm, ids_hbm, out_hbm, ids_v, trel_v, pos_v, stage_a,
               stage_b, rows_v, sem_a, sem_b, sem_o):
    wid = lax.axis_index("s") * NC + lax.axis_index("c")
    rr0 = (wid * R_PER_W) // 128 * 128
    rr1 = jnp.where(wid == NW - 1, VOCAB_N, ((wid + 1) * R_PER_W) // 128 * 128)
    iota = lax.iota(jnp.int32, _LANES)

    def _src0():
        return table3_hbm.at[pl.ds(0, 1), :, pl.ds(rr0, CHUNK)]

    pltpu.async_copy(_src0(), stage_a, sem_a)

    pltpu.sync_copy(ids_hbm, ids_v)

    def prefill(i, _):
        plsc.store_scatter(pos_v, [i * _LANES + iota], _splat(-1))
        return 0

    lax.fori_loop(0, NPOS // _LANES, prefill, 0)

    def scan(i, cursor):
        tok = plsc.load_gather(ids_v, [i * _LANES + iota])
        m = (tok >= rr0) & (tok < rr1)
        mi = m.astype(jnp.int32)
        p = cursor + plsc.cumsum(mi) - 1
        plsc.store_scatter(trel_v, [p], tok - rr0, mask=m)
        plsc.store_scatter(pos_v, [p], i * _LANES + iota, mask=m)
        return cursor + plsc.all_reduce_population_count(m)

    cursor = lax.fori_loop(0, B // _LANES, scan, _splat(0))
    n = jnp.max(cursor)
    waves = jnp.maximum((n + WAVE - 1) // WAVE, 1)

    zeros16 = _splat(0)

    def wave_body(w, _):
        lo = w * WAVE

        NJ = 8 * NRB

        def src(j):
            tr = j // NRB
            r_lo_c = jnp.minimum(rr0 + (j % NRB) * CHUNK, CLAMP_MAX)
            return table3_hbm.at[pl.ds(tr, 1), :, pl.ds(r_lo_c, CHUNK)]

        def fire(j, stage, sem):
            pltpu.async_copy(src(j), stage, sem)

        def gather(j, stage):
            tr = j // NRB
            base_rel = jnp.minimum(rr0 + (j % NRB) * CHUNK, CLAMP_MAX) - rr0
            c0 = tr * 8

            def gv_body(v, _):
                le = plsc.load_gather(trel_v, [lo + v * _LANES + iota])
                idx = le - base_rel
                m = (le >= 0) & (idx >= 0) & (idx < CHUNK)
                idx = jnp.clip(idx, 0, CHUNK - 1)
                il = v * _LANES + iota
                for cp in range(8):
                    vals = plsc.load_gather(
                        stage, [zeros16, _splat(cp), idx], mask=m
                    )
                    plsc.store_scatter(
                        rows_v, [il, jnp.full((_LANES,), c0 + cp, jnp.int32)],
                        vals, mask=m,
                    )
                return 0

            lax.fori_loop(0, WAVE // _LANES, gv_body, 0)

        def wait(j, stage, sem):
            pltpu.make_async_copy(src(j), stage, sem).wait()

        @pl.when(w > 0)
        def _():
            fire(0, stage_a, sem_a)

        def pair_body(k, _):
            fire(2 * k + 1, stage_b, sem_b)
            wait(2 * k, stage_a, sem_a)
            gather(2 * k, stage_a)

            @pl.when(k < NJ // 2 - 1)
            def _():
                fire(2 * k + 2, stage_a, sem_a)

            wait(2 * k + 1, stage_b, sem_b)
            gather(2 * k + 1, stage_b)
            return 0

        lax.fori_loop(0, NJ // 2, pair_body, 0)

        pltpu.async_copy(
            rows_v,
            out_hbm.at[plsc.Indices(pos_v.at[pl.ds(lo, WAVE)], ignored_value=-1)],
            sem_o,
        ).wait()
        return 0

    lax.fori_loop(0, waves, wave_body, 0)


def _proj_body(x_ref, w2_ref, o_ref):
    res = lax.dot_general(
        x_ref[...].astype(jnp.bfloat16), w2_ref[...].astype(jnp.bfloat16),
        dimension_numbers=(((1,), (1,)), ((), ())),
        preferred_element_type=jnp.float32,
    )
    o_ref[...] = res.reshape(ROW_BLK // BATCH, BATCH, D_MODEL)


ROW_BLK = 1024


def _project(x, w2):
    return pl.pallas_call(
        _proj_body,
        grid=(B // ROW_BLK,),
        in_specs=[
            pl.BlockSpec((ROW_BLK, PAIR_W), lambda i: (i, 0)),
            pl.BlockSpec((D_MODEL, PAIR_W), lambda i: (0, 0)),
        ],
        out_specs=pl.BlockSpec(
            (ROW_BLK // BATCH, BATCH, D_MODEL), lambda i: (i, 0, 0)
        ),
        out_shape=jax.ShapeDtypeStruct((SEQ, BATCH, D_MODEL), jnp.float32),
    )(x, w2)


def kernel(input_ids, emb_table, emb_proj):
    ids = jnp.transpose(input_ids, (1, 0)).astype(jnp.int32).reshape(B)
    table3 = jnp.transpose(emb_table, (1, 0)).reshape(8, 8, VOCAB_N)
    w2 = jnp.concatenate(
        [emb_proj, jnp.zeros((D_MODEL, D_EMBED), jnp.float32)], axis=1
    )
    rows = _sc_gather(table3, ids)
    return _project(rows, w2)


# speedup vs baseline: 1.2943x; 1.0065x over previous
import functools

import jax
import jax.numpy as jnp
from jax import lax
from jax.experimental import pallas as pl
from jax.experimental.pallas import tpu as pltpu
from jax.experimental.pallas import tpu_sc as plsc

D_EMBED = 64
D_MODEL = 1024
BATCH = 4
SEQ = 2048
B = BATCH * SEQ
VOCAB_N = 1000000
PAIR_W = 128

NC, NS = 2, 16
NW = NC * NS
R_PER_W = VOCAB_N // NW
CHUNK = 2688
NRB = 12
LIMIT = 1000064
CLAMP_MAX = (LIMIT - CHUNK) // 128 * 128
WAVE = 384
NPOS = ((B + WAVE - 1) // WAVE) * WAVE

_mesh = plsc.VectorSubcoreMesh(
    core_axis_name="c", subcore_axis_name="s", num_cores=NC, num_subcores=NS
)

_LANES = 16


def _splat(x):
    return jnp.full((_LANES,), x, jnp.int32)


@functools.partial(
    pl.kernel,
    out_type=jax.ShapeDtypeStruct((B, PAIR_W), jnp.float32),
    mesh=_mesh,
    scratch_types=[
        pltpu.VMEM((B,), jnp.int32),
        pltpu.VMEM((B,), jnp.int32),
        pltpu.VMEM((NPOS,), jnp.int32),
        pltpu.VMEM((1, 8, CHUNK), jnp.float32),
        pltpu.VMEM((1, 8, CHUNK), jnp.float32),
        pltpu.VMEM((WAVE, PAIR_W), jnp.float32),
        pltpu.SemaphoreType.DMA,
        pltpu.SemaphoreType.DMA,
        pltpu.SemaphoreType.DMA,
    ],
    compiler_params=pltpu.CompilerParams(
        use_tc_tiling_on_sc=True,
        disable_bounds_checks=True,
        needs_layout_passes=False,
    ),
)
def _sc_gather(table3_hbm, ids_hbm, out_hbm, ids_v, trel_v, pos_v, stage_a,
               stage_b, rows_v, sem_a, sem_b, sem_o):
    wid = lax.axis_index("s") * NC + lax.axis_index("c")
    rr0 = (wid * R_PER_W) // 128 * 128
    rr1 = jnp.where(wid == NW - 1, VOCAB_N, ((wid + 1) * R_PER_W) // 128 * 128)
    iota = lax.iota(jnp.int32, _LANES)

    def _src0():
        return table3_hbm.at[pl.ds(0, 1), :, pl.ds(rr0, CHUNK)]

    pltpu.async_copy(_src0(), stage_a, sem_a)

    pltpu.sync_copy(ids_hbm, ids_v)

    def prefill(i, _):
        plsc.store_scatter(pos_v, [i * _LANES + iota], _splat(-1))
        return 0

    lax.fori_loop(0, NPOS // _LANES, prefill, 0)

    def scan(i, cursor):
        tok = plsc.load_gather(ids_v, [i * _LANES + iota])
        m = (tok >= rr0) & (tok < rr1)
        mi = m.astype(jnp.int32)
        p = cursor + plsc.cumsum(mi) - 1
        plsc.store_scatter(trel_v, [p], tok - rr0, mask=m)
        plsc.store_scatter(pos_v, [p], i * _LANES + iota, mask=m)
        return cursor + plsc.all_reduce_population_count(m)

    cursor = lax.fori_loop(0, B // _LANES, scan, _splat(0))
    n = jnp.max(cursor)
    waves = jnp.maximum((n + WAVE - 1) // WAVE, 1)

    zeros16 = _splat(0)

    def wave_body(w, _):
        lo = w * WAVE

        NJ = 8 * NRB

        def src(j):
            tr = j // NRB
            r_lo_c = jnp.minimum(rr0 + (j % NRB) * CHUNK, CLAMP_MAX)
            return table3_hbm.at[pl.ds(tr, 1), :, pl.ds(r_lo_c, CHUNK)]

        def fire(j, stage, sem):
            pltpu.async_copy(src(j), stage, sem)

        def gather(j, stage):
            tr = j // NRB
            base_rel = jnp.minimum(rr0 + (j % NRB) * CHUNK, CLAMP_MAX) - rr0
            c0 = tr * 8

            def gv_body(v, _):
                le = plsc.load_gather(trel_v, [lo + v * _LANES + iota])
                idx = le - base_rel
                m = (le >= 0) & (idx >= 0) & (idx < CHUNK)
                idx = jnp.clip(idx, 0, CHUNK - 1)
                il = v * _LANES + iota
                for cp in range(8):
                    vals = plsc.load_gather(
                        stage, [zeros16, _splat(cp), idx], mask=m
                    )
                    plsc.store_scatter(
                        rows_v, [il, jnp.full((_LANES,), c0 + cp, jnp.int32)],
                        vals, mask=m,
                    )
                return 0

            lax.fori_loop(0, WAVE // _LANES, gv_body, 0)

        def wait(j, stage, sem):
            pltpu.make_async_copy(src(j), stage, sem).wait()

        @pl.when(w > 0)
        def _():
            fire(0, stage_a, sem_a)

        def pair_body(k, _):
            fire(2 * k + 1, stage_b, sem_b)
            wait(2 * k, stage_a, sem_a)
            gather(2 * k, stage_a)

            @pl.when(k < NJ // 2 - 1)
            def _():
                fire(2 * k + 2, stage_a, sem_a)

            wait(2 * k + 1, stage_b, sem_b)
            gather(2 * k + 1, stage_b)
            return 0

        lax.fori_loop(0, NJ // 2, pair_body, 0)

        pltpu.async_copy(
            rows_v,
            out_hbm.at[plsc.Indices(pos_v.at[pl.ds(lo, WAVE)], ignored_value=-1)],
            sem_o,
        ).wait()
        return 0

    lax.fori_loop(0, waves, wave_body, 0)


def _proj_body(x_ref, w2_ref, o_ref):
    res = lax.dot_general(
        x_ref[...].astype(jnp.bfloat16), w2_ref[...].astype(jnp.bfloat16),
        dimension_numbers=(((1,), (1,)), ((), ())),
        preferred_element_type=jnp.float32,
    )
    o_ref[...] = res.reshape(ROW_BLK // BATCH, BATCH, D_MODEL)


ROW_BLK = 2048


def _project(x, w2):
    return pl.pallas_call(
        _proj_body,
        grid=(B // ROW_BLK,),
        in_specs=[
            pl.BlockSpec((ROW_BLK, PAIR_W), lambda i: (i, 0)),
            pl.BlockSpec((D_MODEL, PAIR_W), lambda i: (0, 0)),
        ],
        out_specs=pl.BlockSpec(
            (ROW_BLK // BATCH, BATCH, D_MODEL), lambda i: (i, 0, 0)
        ),
        out_shape=jax.ShapeDtypeStruct((SEQ, BATCH, D_MODEL), jnp.float32),
    )(x, w2)


def kernel(input_ids, emb_table, emb_proj):
    ids = jnp.transpose(input_ids, (1, 0)).astype(jnp.int32).reshape(B)
    table3 = jnp.transpose(emb_table, (1, 0)).reshape(8, 8, VOCAB_N)
    w2 = jnp.concatenate(
        [emb_proj, jnp.zeros((D_MODEL, D_EMBED), jnp.float32)], axis=1
    )
    rows = _sc_gather(table3, ids)
    return _project(rows, w2)


# speedup vs baseline: 1.2976x; 1.0025x over previous
import functools

import jax
import jax.numpy as jnp
from jax import lax
from jax.experimental import pallas as pl
from jax.experimental.pallas import tpu as pltpu
from jax.experimental.pallas import tpu_sc as plsc

D_EMBED = 64
D_MODEL = 1024
BATCH = 4
SEQ = 2048
B = BATCH * SEQ
VOCAB_N = 1000000
PAIR_W = 128

NC, NS = 2, 16
NW = NC * NS
R_PER_W = VOCAB_N // NW
CHUNK = 2688
NRB = 12
LIMIT = 1000064
CLAMP_MAX = (LIMIT - CHUNK) // 128 * 128
WAVE = 384
NPOS = ((B + WAVE - 1) // WAVE) * WAVE

_mesh = plsc.VectorSubcoreMesh(
    core_axis_name="c", subcore_axis_name="s", num_cores=NC, num_subcores=NS
)

_LANES = 16


def _splat(x):
    return jnp.full((_LANES,), x, jnp.int32)


@functools.partial(
    pl.kernel,
    out_type=jax.ShapeDtypeStruct((B, PAIR_W), jnp.float32),
    mesh=_mesh,
    scratch_types=[
        pltpu.VMEM((B,), jnp.int32),
        pltpu.VMEM((NPOS,), jnp.int32),
        pltpu.VMEM((NPOS,), jnp.int32),
        pltpu.VMEM((1, 8, CHUNK), jnp.float32),
        pltpu.VMEM((1, 8, CHUNK), jnp.float32),
        pltpu.VMEM((WAVE, PAIR_W), jnp.float32),
        pltpu.SemaphoreType.DMA,
        pltpu.SemaphoreType.DMA,
        pltpu.SemaphoreType.DMA,
    ],
    compiler_params=pltpu.CompilerParams(
        use_tc_tiling_on_sc=True,
        disable_bounds_checks=True,
        needs_layout_passes=False,
    ),
)
def _sc_gather(table3_hbm, ids_hbm, out_hbm, ids_v, trel_v, pos_v, stage_a,
               stage_b, rows_v, sem_a, sem_b, sem_o):
    wid = lax.axis_index("s") * NC + lax.axis_index("c")
    rr0 = (wid * R_PER_W) // 128 * 128
    rr1 = jnp.where(wid == NW - 1, VOCAB_N, ((wid + 1) * R_PER_W) // 128 * 128)
    iota = lax.iota(jnp.int32, _LANES)

    def _src0():
        return table3_hbm.at[pl.ds(0, 1), :, pl.ds(rr0, CHUNK)]

    pltpu.async_copy(_src0(), stage_a, sem_a)

    pltpu.sync_copy(ids_hbm, ids_v)

    def prefill(i, _):
        plsc.store_scatter(pos_v, [i * _LANES + iota], _splat(-1))
        return 0

    lax.fori_loop(0, NPOS // _LANES, prefill, 0)

    def scan(i, cursor):
        tok = plsc.load_gather(ids_v, [i * _LANES + iota])
        m = (tok >= rr0) & (tok < rr1)
        mi = m.astype(jnp.int32)
        p = cursor + plsc.cumsum(mi) - 1
        plsc.store_scatter(trel_v, [p], tok - rr0, mask=m)
        plsc.store_scatter(pos_v, [p], i * _LANES + iota, mask=m)
        return cursor + plsc.all_reduce_population_count(m)

    cursor = lax.fori_loop(0, B // _LANES, scan, _splat(0))
    n = jnp.max(cursor)
    waves = jnp.maximum((n + WAVE - 1) // WAVE, 1)

    zeros16 = _splat(0)

    def wave_body(w, _):
        lo = w * WAVE

        NJ = 8 * NRB

        def src(j):
            tr = j // NRB
            r_lo_c = jnp.minimum(rr0 + (j % NRB) * CHUNK, CLAMP_MAX)
            return table3_hbm.at[pl.ds(tr, 1), :, pl.ds(r_lo_c, CHUNK)]

        def fire(j, stage, sem):
            pltpu.async_copy(src(j), stage, sem)

        def gather(j, stage):
            tr = j // NRB
            base_rel = jnp.minimum(rr0 + (j % NRB) * CHUNK, CLAMP_MAX) - rr0
            c0 = tr * 8

            def gv_body(v, _):
                le = plsc.load_gather(trel_v, [lo + v * _LANES + iota])
                idx = le - base_rel
                m = (le >= 0) & (idx >= 0) & (idx < CHUNK)
                idx = jnp.clip(idx, 0, CHUNK - 1)
                il = v * _LANES + iota
                for cp in range(8):
                    vals = plsc.load_gather(
                        stage, [zeros16, _splat(cp), idx], mask=m
                    )
                    plsc.store_scatter(
                        rows_v, [il, jnp.full((_LANES,), c0 + cp, jnp.int32)],
                        vals, mask=m,
                    )
                return 0

            lax.fori_loop(0, WAVE // _LANES, gv_body, 0)

        def wait(j, stage, sem):
            pltpu.make_async_copy(src(j), stage, sem).wait()

        @pl.when(w > 0)
        def _():
            fire(0, stage_a, sem_a)

        def pair_body(k, _):
            fire(2 * k + 1, stage_b, sem_b)
            wait(2 * k, stage_a, sem_a)
            gather(2 * k, stage_a)

            @pl.when(k < NJ // 2 - 1)
            def _():
                fire(2 * k + 2, stage_a, sem_a)

            wait(2 * k + 1, stage_b, sem_b)
            gather(2 * k + 1, stage_b)
            return 0

        lax.fori_loop(0, NJ // 2, pair_body, 0)

        pltpu.async_copy(
            rows_v,
            out_hbm.at[plsc.Indices(pos_v.at[pl.ds(lo, WAVE)], ignored_value=-1)],
            sem_o,
        ).wait()
        return 0

    lax.fori_loop(0, waves, wave_body, 0)


def _proj_body(x_ref, w2_ref, o_ref):
    res = lax.dot_general(
        x_ref[...].astype(jnp.bfloat16), w2_ref[...].astype(jnp.bfloat16),
        dimension_numbers=(((1,), (1,)), ((), ())),
        preferred_element_type=jnp.float32,
    )
    o_ref[...] = res.reshape(ROW_BLK // BATCH, BATCH, D_MODEL)


ROW_BLK = 2048


def _project(x, w2):
    return pl.pallas_call(
        _proj_body,
        grid=(B // ROW_BLK,),
        in_specs=[
            pl.BlockSpec((ROW_BLK, PAIR_W), lambda i: (i, 0)),
            pl.BlockSpec((D_MODEL, PAIR_W), lambda i: (0, 0)),
        ],
        out_specs=pl.BlockSpec(
            (ROW_BLK // BATCH, BATCH, D_MODEL), lambda i: (i, 0, 0)
        ),
        out_shape=jax.ShapeDtypeStruct((SEQ, BATCH, D_MODEL), jnp.float32),
    )(x, w2)


def kernel(input_ids, emb_table, emb_proj):
    ids = jnp.transpose(input_ids, (1, 0)).astype(jnp.int32).reshape(B)
    table3 = jnp.transpose(emb_table, (1, 0)).reshape(8, 8, VOCAB_N)
    w2 = jnp.concatenate(
        [emb_proj, jnp.zeros((D_MODEL, D_EMBED), jnp.float32)], axis=1
    )
    rows = _sc_gather(table3, ids)
    return _project(rows, w2)
